# Initial kernel scaffold; baseline (speedup 1.0000x reference)
import jax
import jax.numpy as jnp
from jax.experimental import pallas as pl


def kernel(row_indices, col_indices, row_pe, col_pe):
    raise NotImplementedError("write your pallas kernel here")


# speedup vs baseline: 15.7955x; 15.7955x over previous
import functools

import jax
import jax.numpy as jnp
from jax import lax
from jax.experimental import pallas as pl
from jax.experimental.pallas import tpu as pltpu
from jax.experimental.pallas import tpu_sc as plsc

_NC = 2
_NS = 16
_NW = _NC * _NS
_L = 16

_CHUNK = 512
_GATHER = 128


def _make_sc_gather(B, D, n_rows, n_cols):
    W = B // _NW
    n_chunks = W // _CHUNK

    mesh = plsc.VectorSubcoreMesh(core_axis_name="c", subcore_axis_name="s")

    @functools.partial(
        pl.kernel,
        out_type=jax.ShapeDtypeStruct((B, D), jnp.float32),
        mesh=mesh,
        scratch_types=[
            pltpu.VMEM((_CHUNK,), jnp.int32),
            pltpu.VMEM((_CHUNK,), jnp.int32),
            pltpu.VMEM((_CHUNK,), jnp.int32),
            pltpu.VMEM((_CHUNK, D), jnp.float32),
            pltpu.SemaphoreType.DMA,
        ],
    )
    def k(table_hbm, ri_hbm, ci_hbm, out_hbm, ri_v, ci_v, fi_v, rows_v, sem):
        wid = lax.axis_index("s") * _NC + lax.axis_index("c")
        base = wid * W

        def body(i, carry):
            off = base + i * _CHUNK
            pltpu.sync_copy(ri_hbm.at[pl.ds(off, _CHUNK)], ri_v)
            pltpu.sync_copy(ci_hbm.at[pl.ds(off, _CHUNK)], ci_v)

            def fuse(t, c):
                sl = pl.ds(t * _L, _L)
                r = jnp.clip(ri_v[sl], 0, n_rows - 1)
                cc = jnp.clip(ci_v[sl], 0, n_cols - 1)
                fi_v[sl] = r * n_cols + cc
                return c

            lax.fori_loop(0, _CHUNK // _L, fuse, 0, unroll=4)

            copies = [
                pltpu.async_copy(
                    table_hbm.at[fi_v.at[pl.ds(g * _GATHER, _GATHER)]],
                    rows_v.at[pl.ds(g * _GATHER, _GATHER)],
                    sem,
                )
                for g in range(_CHUNK // _GATHER)
            ]
            for cp in copies:
                cp.wait()
            pltpu.sync_copy(rows_v, out_hbm.at[pl.ds(off, _CHUNK)])
            return carry

        lax.fori_loop(0, n_chunks, body, 0)

    return k


def kernel(row_indices, col_indices, row_pe, col_pe):
    R, Dr = row_pe.shape
    C, Dc = col_pe.shape
    D = Dr + Dc
    shp = row_indices.shape
    B = row_indices.size

    fused_table = jnp.concatenate(
        [
            jnp.broadcast_to(row_pe[:, None, :], (R, C, Dr)),
            jnp.broadcast_to(col_pe[None, :, :], (R, C, Dc)),
        ],
        axis=-1,
    ).reshape(R * C, D)

    ri = row_indices.reshape(B)
    ci = col_indices.reshape(B)

    out = _make_sc_gather(B, D, R, C)(fused_table, ri, ci)
    return out.reshape(shp + (D,))


# speedup vs baseline: 17.8181x; 1.1280x over previous
import functools

import jax
import jax.numpy as jnp
from jax import lax
from jax.experimental import pallas as pl
from jax.experimental.pallas import tpu as pltpu
from jax.experimental.pallas import tpu_sc as plsc

_NC = 2
_NS = 16
_NW = _NC * _NS
_L = 16

_CHUNK = 256
_GATHER = 128
_IDXBLK = 1600


def _make_sc_gather(B, D, n_rows, n_cols):
    W = B // _NW
    n_chunks = W // _CHUNK
    n_blks = W // _IDXBLK
    assert W % _IDXBLK == 0 and W % _CHUNK == 0
    assert _CHUNK % _GATHER == 0 and n_chunks % 2 == 0 and n_chunks >= 4

    mesh = plsc.VectorSubcoreMesh(core_axis_name="c", subcore_axis_name="s")

    @functools.partial(
        pl.kernel,
        out_type=jax.ShapeDtypeStruct((B, D), jnp.float32),
        mesh=mesh,
        scratch_types=[
            pltpu.VMEM((_IDXBLK,), jnp.int32),
            pltpu.VMEM((_IDXBLK,), jnp.int32),
            pltpu.VMEM((W,), jnp.int32),
            pltpu.VMEM((_CHUNK, D), jnp.float32),
            pltpu.VMEM((_CHUNK, D), jnp.float32),
            pltpu.SemaphoreType.DMA,
            pltpu.SemaphoreType.DMA,
            pltpu.SemaphoreType.DMA,
            pltpu.SemaphoreType.DMA,
        ],
    )
    def k(table_hbm, ri_hbm, ci_hbm, out_hbm,
          ri_v, ci_v, fi_v, rows0, rows1, gsem0, gsem1, wsem0, wsem1):
        wid = lax.axis_index("s") * _NC + lax.axis_index("c")
        base = wid * W
        rows = (rows0, rows1)
        gsem = (gsem0, gsem1)
        wsem = (wsem0, wsem1)

        def blk_body(blk, carry):
            off = blk * _IDXBLK
            pltpu.sync_copy(ri_hbm.at[pl.ds(base + off, _IDXBLK)], ri_v)
            pltpu.sync_copy(ci_hbm.at[pl.ds(base + off, _IDXBLK)], ci_v)

            def fuse(t, c):
                sl = pl.ds(t * _L, _L)
                r = jnp.clip(ri_v[sl], 0, n_rows - 1)
                cc = jnp.clip(ci_v[sl], 0, n_cols - 1)
                fi_v[pl.ds(off + t * _L, _L)] = r * n_cols + cc
                return c

            lax.fori_loop(0, _IDXBLK // _L, fuse, 0, unroll=4)
            return carry

        lax.fori_loop(0, n_blks, blk_body, 0)

        def g_fire(i, b):
            for g in range(_CHUNK // _GATHER):
                pltpu.async_copy(
                    table_hbm.at[fi_v.at[pl.ds(i * _CHUNK + g * _GATHER, _GATHER)]],
                    rows[b].at[pl.ds(g * _GATHER, _GATHER)],
                    gsem[b],
                )

        def g_wait(i, b):
            for g in range(_CHUNK // _GATHER):
                pltpu.make_async_copy(
                    table_hbm.at[fi_v.at[pl.ds(i * _CHUNK + g * _GATHER, _GATHER)]],
                    rows[b].at[pl.ds(g * _GATHER, _GATHER)],
                    gsem[b],
                ).wait()

        def w_fire(i, b):
            pltpu.async_copy(rows[b], out_hbm.at[pl.ds(base + i * _CHUNK, _CHUNK)], wsem[b])

        def w_wait(b):
            pltpu.make_async_copy(
                rows[b], out_hbm.at[pl.ds(base, _CHUNK)], wsem[b]
            ).wait()

        g_fire(0, 0)
        g_wait(0, 0)
        w_fire(0, 0)
        g_fire(1, 1)

        def steady(kk, carry):
            i0 = 1 + 2 * kk
            for d in range(2):
                i = i0 + d
                b = (1 + d) % 2
                nb = 1 - b
                g_wait(i, b)
                w_fire(i, b)
                w_wait(nb)
                g_fire(i + 1, nb)
            return carry

        lax.fori_loop(0, (n_chunks - 2) // 2, steady, 0)

        g_wait(n_chunks - 1, 1)
        w_fire(n_chunks - 1, 1)
        w_wait(0)
        w_wait(1)

    return k


def kernel(row_indices, col_indices, row_pe, col_pe):
    R, Dr = row_pe.shape
    C, Dc = col_pe.shape
    D = Dr + Dc
    shp = row_indices.shape
    B = row_indices.size

    fused_table = jnp.concatenate(
        [
            jnp.broadcast_to(row_pe[:, None, :], (R, C, Dr)),
            jnp.broadcast_to(col_pe[None, :, :], (R, C, Dc)),
        ],
        axis=-1,
    ).reshape(R * C, D)

    ri = row_indices.reshape(B)
    ci = col_indices.reshape(B)

    out = _make_sc_gather(B, D, R, C)(fused_table, ri, ci)
    return out.reshape(shp + (D,))


# speedup vs baseline: 17.9902x; 1.0097x over previous
import functools

import jax
import jax.numpy as jnp
from jax import lax
from jax.experimental import pallas as pl
from jax.experimental.pallas import tpu as pltpu
from jax.experimental.pallas import tpu_sc as plsc

_NC = 2
_NS = 16
_NW = _NC * _NS
_L = 16

_CHUNK = 128
_NSLOT = 4
_IDXBLK = 1600


def _make_sc_gather(B, D, n_rows, n_cols):
    W = B // _NW
    n_chunks = W // _CHUNK
    n_blks = W // _IDXBLK
    assert W % _IDXBLK == 0 and W % _CHUNK == 0
    assert (n_chunks - _NSLOT) % _NSLOT == 0 and n_chunks >= 2 * _NSLOT

    mesh = plsc.VectorSubcoreMesh(core_axis_name="c", subcore_axis_name="s")

    @functools.partial(
        pl.kernel,
        out_type=jax.ShapeDtypeStruct((B, D), jnp.float32),
        mesh=mesh,
        scratch_types=[
            pltpu.VMEM((_IDXBLK,), jnp.int32),
            pltpu.VMEM((_IDXBLK,), jnp.int32),
            pltpu.VMEM((W,), jnp.int32),
        ] + [pltpu.VMEM((_CHUNK, D), jnp.float32) for _ in range(_NSLOT)]
          + [pltpu.SemaphoreType.DMA for _ in range(2 * _NSLOT)],
    )
    def k(table_hbm, ri_hbm, ci_hbm, out_hbm, ri_v, ci_v, fi_v, *slots):
        wid = lax.axis_index("s") * _NC + lax.axis_index("c")
        base = wid * W
        rows = slots[:_NSLOT]
        gsem = slots[_NSLOT:2 * _NSLOT]
        wsem = slots[2 * _NSLOT:]

        def blk_body(blk, carry):
            off = blk * _IDXBLK
            pltpu.sync_copy(ri_hbm.at[pl.ds(base + off, _IDXBLK)], ri_v)
            pltpu.sync_copy(ci_hbm.at[pl.ds(base + off, _IDXBLK)], ci_v)

            def fuse(t, c):
                sl = pl.ds(t * _L, _L)
                r = jnp.clip(ri_v[sl], 0, n_rows - 1)
                cc = jnp.clip(ci_v[sl], 0, n_cols - 1)
                fi_v[pl.ds(off + t * _L, _L)] = r * n_cols + cc
                return c

            lax.fori_loop(0, _IDXBLK // _L, fuse, 0, unroll=4)
            return carry

        lax.fori_loop(0, n_blks, blk_body, 0)

        def g_fire(i, b):
            pltpu.async_copy(
                table_hbm.at[fi_v.at[pl.ds(i * _CHUNK, _CHUNK)]],
                rows[b],
                gsem[b],
            )

        def g_wait(i, b):
            pltpu.make_async_copy(
                table_hbm.at[fi_v.at[pl.ds(i * _CHUNK, _CHUNK)]],
                rows[b],
                gsem[b],
            ).wait()

        def w_fire(i, b):
            pltpu.async_copy(rows[b], out_hbm.at[pl.ds(base + i * _CHUNK, _CHUNK)], wsem[b])

        def w_wait(b):
            pltpu.make_async_copy(
                rows[b], out_hbm.at[pl.ds(base, _CHUNK)], wsem[b]
            ).wait()

        for j in range(_NSLOT - 1):
            g_fire(j, j)
        g_wait(0, 0)
        w_fire(0, 0)
        g_fire(_NSLOT - 1, _NSLOT - 1)

        n_steady = n_chunks - _NSLOT
        assert n_steady % _NSLOT == 0

        def steady(kk, carry):
            i0 = 1 + _NSLOT * kk
            for d in range(_NSLOT):
                i = i0 + d
                b = (1 + d) % _NSLOT
                pb = d % _NSLOT
                g_wait(i, b)
                w_fire(i, b)
                w_wait(pb)
                g_fire(i + _NSLOT - 1, pb)
            return carry

        lax.fori_loop(0, n_steady // _NSLOT, steady, 0)

        for j in range(_NSLOT - 1, 0, -1):
            i = n_chunks - j
            b = i % _NSLOT
            g_wait(i, b)
            w_fire(i, b)
            w_wait((i - 1) % _NSLOT)
        w_wait((n_chunks - 1) % _NSLOT)

    return k


def kernel(row_indices, col_indices, row_pe, col_pe):
    R, Dr = row_pe.shape
    C, Dc = col_pe.shape
    D = Dr + Dc
    shp = row_indices.shape
    B = row_indices.size

    fused_table = jnp.concatenate(
        [
            jnp.broadcast_to(row_pe[:, None, :], (R, C, Dr)),
            jnp.broadcast_to(col_pe[None, :, :], (R, C, Dc)),
        ],
        axis=-1,
    ).reshape(R * C, D)

    ri = row_indices.reshape(B)
    ci = col_indices.reshape(B)

    out = _make_sc_gather(B, D, R, C)(fused_table, ri, ci)
    return out.reshape(shp + (D,))
